# baseline (speedup 1.0000x reference)
import functools

import jax
import jax.numpy as jnp
from jax.experimental import pallas as pl
from jax.experimental.pallas import tpu as pltpu

_K = 8
_MIN_NEG = -1000000000.0
_REMOVED = -2.0e9
_HI = jax.lax.Precision.HIGHEST


def _body(h_ref, m_ref, c_ref, score_ref, emb_ref, s_ref, *, bb, L, C, D):
    CP = 64
    ones_d = jnp.ones((D, 1), dtype=jnp.float32)
    hns = []
    for i in range(bb):
        h = h_ref[i]
        c = c_ref[i]

        hn2 = jax.lax.dot_general(h * h, ones_d, (((1,), (0,)), ((), ())),
                                  precision=_HI,
                                  preferred_element_type=jnp.float32)
        hn = h * jax.lax.rsqrt(jnp.maximum(hn2, 1e-24))
        cn2 = jax.lax.dot_general(c * c, ones_d, (((1,), (0,)), ((), ())),
                                  precision=_HI,
                                  preferred_element_type=jnp.float32)
        cn = c * jax.lax.rsqrt(jnp.maximum(cn2, 1e-24))
        hns.append(hn)

        st = jax.lax.dot_general(hn, cn, (((1,), (1,)), ((), ())),
                                 preferred_element_type=jnp.float32)
        s_ref[:, pl.ds(i * CP, C)] = st

    pm = (m_ref[0] - 1.0) * (-_MIN_NEG)
    sel_i = jax.lax.broadcasted_iota(jnp.int32, (bb, bb * CP), 1) // CP
    sel_j = jax.lax.broadcasted_iota(jnp.int32, (bb, bb * CP), 0)
    esel = jnp.where(sel_i == sel_j, 1.0, 0.0)
    pen = jax.lax.dot_general(pm, esel, (((1,), (0,)), ((), ())),
                              preferred_element_type=jnp.float32)

    sm0 = s_ref[:, :] + pen

    m = jnp.max(sm0, axis=0, keepdims=True)
    for _ in range(_K - 1):
        m = jnp.max(jnp.where(sm0 < m, sm0, _REMOVED), axis=0, keepdims=True)
    t8 = m

    w = jnp.where((sm0 >= t8) & (sm0 > (_MIN_NEG * 0.5)), 1.0, 0.0)

    ones_l = jnp.ones((1, L), dtype=jnp.float32)
    ssum = jax.lax.dot_general(ones_l, w * sm0, (((1,), (0,)), ((), ())),
                               preferred_element_type=jnp.float32)
    cnt = jax.lax.dot_general(ones_l, w, (((1,), (0,)), ((), ())),
                              preferred_element_type=jnp.float32)
    inv = 1.0 / jnp.maximum(cnt, 1.0)
    score = ssum * inv
    w2 = w * inv

    for i in range(bb):
        score_ref[pl.ds(i, 1), :] = score[:, i * CP:i * CP + C]
        wi = w2[:, i * CP:i * CP + C]
        g = jax.lax.dot_general(wi, hns[i], (((0,), (0,)), ((), ())),
                                preferred_element_type=jnp.float32)
        emb_ref[i] = g


def kernel(hist_item_emb, hist_item_mask, cand_item_emb):
    B, L, D = hist_item_emb.shape
    C = cand_item_emb.shape[1]
    bb = 8
    CP = 64

    body = functools.partial(_body, bb=bb, L=L, C=C, D=D)

    mask_t = jnp.transpose(
        hist_item_mask.reshape(B // bb, bb, L), (0, 2, 1)
    ).astype(jnp.float32)

    out = pl.pallas_call(
        body,
        grid=(B // bb,),
        in_specs=[
            pl.BlockSpec((bb, L, D), lambda i: (i, 0, 0)),
            pl.BlockSpec((1, L, bb), lambda i: (i, 0, 0)),
            pl.BlockSpec((bb, C, D), lambda i: (i, 0, 0)),
        ],
        out_specs=[
            pl.BlockSpec((bb, C), lambda i: (i, 0)),
            pl.BlockSpec((bb, C, D), lambda i: (i, 0, 0)),
        ],
        out_shape=[
            jax.ShapeDtypeStruct((B, C), jnp.float32),
            jax.ShapeDtypeStruct((B, C, D), jnp.float32),
        ],
        scratch_shapes=[pltpu.VMEM((L, bb * CP), jnp.float32)],
    )(hist_item_emb, mask_t, cand_item_emb)
    return (out[0], out[1])


# speedup vs baseline: 2.4868x; 2.4868x over previous
import functools

import jax
import jax.numpy as jnp
from jax.experimental import pallas as pl
from jax.experimental.pallas import tpu as pltpu

_K = 8
_MIN_NEG = -1000000000.0
_REMOVED = -2.0e9
_HI = jax.lax.Precision.HIGHEST


def _body(h_ref, m_ref, c_ref, score_ref, emb_ref, s_ref, *, bb, L, C, D):
    CP = 64
    ones_d = jnp.ones((D, 1), dtype=jnp.float32)
    hns = []
    for i in range(bb):
        h = h_ref[i]
        c = c_ref[i]

        hn2 = jnp.sum(h * h, axis=1, keepdims=True)
        hn = h * jax.lax.rsqrt(jnp.maximum(hn2, 1e-24))
        cn2 = jnp.sum(c * c, axis=1, keepdims=True)
        cn = c * jax.lax.rsqrt(jnp.maximum(cn2, 1e-24))
        hns.append(hn)

        st = jax.lax.dot_general(hn, cn, (((1,), (1,)), ((), ())),
                                 preferred_element_type=jnp.float32)
        s_ref[:, pl.ds(i * CP, C)] = st

    pm = (m_ref[0] - 1.0) * (-_MIN_NEG)
    sel_i = jax.lax.broadcasted_iota(jnp.int32, (bb, bb * CP), 1) // CP
    sel_j = jax.lax.broadcasted_iota(jnp.int32, (bb, bb * CP), 0)
    esel = jnp.where(sel_i == sel_j, 1.0, 0.0)
    pen = jax.lax.dot_general(pm, esel, (((1,), (0,)), ((), ())),
                              preferred_element_type=jnp.float32)

    sm0 = s_ref[:, :] + pen

    m = jnp.max(sm0, axis=0, keepdims=True)
    for _ in range(_K - 1):
        m = jnp.max(jnp.where(sm0 < m, sm0, _REMOVED), axis=0, keepdims=True)
    t8 = m

    w = jnp.where((sm0 >= t8) & (sm0 > (_MIN_NEG * 0.5)), 1.0, 0.0)

    ones_l = jnp.ones((1, L), dtype=jnp.float32)
    ssum = jax.lax.dot_general(ones_l, w * sm0, (((1,), (0,)), ((), ())),
                               preferred_element_type=jnp.float32)
    cnt = jax.lax.dot_general(ones_l, w, (((1,), (0,)), ((), ())),
                              preferred_element_type=jnp.float32)
    inv = 1.0 / jnp.maximum(cnt, 1.0)
    score = ssum * inv
    w2 = w * inv

    for i in range(bb):
        score_ref[pl.ds(i, 1), :] = score[:, i * CP:i * CP + C]
        wi = w2[:, i * CP:i * CP + C]
        g = jax.lax.dot_general(wi, hns[i], (((0,), (0,)), ((), ())),
                                preferred_element_type=jnp.float32)
        emb_ref[i] = g


def kernel(hist_item_emb, hist_item_mask, cand_item_emb):
    B, L, D = hist_item_emb.shape
    C = cand_item_emb.shape[1]
    bb = 8
    CP = 64

    body = functools.partial(_body, bb=bb, L=L, C=C, D=D)

    mask_t = jnp.transpose(
        hist_item_mask.reshape(B // bb, bb, L), (0, 2, 1)
    ).astype(jnp.float32)

    out = pl.pallas_call(
        body,
        grid=(B // bb,),
        in_specs=[
            pl.BlockSpec((bb, L, D), lambda i: (i, 0, 0)),
            pl.BlockSpec((1, L, bb), lambda i: (i, 0, 0)),
            pl.BlockSpec((bb, C, D), lambda i: (i, 0, 0)),
        ],
        out_specs=[
            pl.BlockSpec((bb, C), lambda i: (i, 0)),
            pl.BlockSpec((bb, C, D), lambda i: (i, 0, 0)),
        ],
        out_shape=[
            jax.ShapeDtypeStruct((B, C), jnp.float32),
            jax.ShapeDtypeStruct((B, C, D), jnp.float32),
        ],
        scratch_shapes=[pltpu.VMEM((L, bb * CP), jnp.float32)],
    )(hist_item_emb, mask_t, cand_item_emb)
    return (out[0], out[1])


# speedup vs baseline: 2.5376x; 1.0204x over previous
import functools

import jax
import jax.numpy as jnp
from jax.experimental import pallas as pl
from jax.experimental.pallas import tpu as pltpu

_K = 8
_MIN_NEG = -1000000000.0
_REMOVED = -2.0e9
_HI = jax.lax.Precision.HIGHEST


def _body(h_ref, m_ref, c_ref, score_ref, emb_ref, s_ref, *, bb, L, C, D):
    CP = 64
    ones_d = jnp.ones((D, 1), dtype=jnp.float32)
    hns = []
    for i in range(bb):
        h = h_ref[i]
        c = c_ref[i]

        hn2 = jnp.sum(h * h, axis=1, keepdims=True)
        hn = h * jax.lax.rsqrt(jnp.maximum(hn2, 1e-24))
        cn2 = jnp.sum(c * c, axis=1, keepdims=True)
        cn = c * jax.lax.rsqrt(jnp.maximum(cn2, 1e-24))
        hns.append(hn)

        st = jax.lax.dot_general(hn, cn, (((1,), (1,)), ((), ())),
                                 preferred_element_type=jnp.float32)
        s_ref[:, pl.ds(i * CP, C)] = jnp.where(m_ref[i].reshape(L, 1) > 0, st, _MIN_NEG)

    sm0 = s_ref[:, :]

    m = jnp.max(sm0, axis=0, keepdims=True)
    for _ in range(_K - 1):
        m = jnp.max(jnp.where(sm0 < m, sm0, _REMOVED), axis=0, keepdims=True)
    t8 = jnp.maximum(m, _MIN_NEG * 0.5)

    w = jnp.where(sm0 >= t8, 1.0, 0.0)

    ones_l = jnp.ones((1, L), dtype=jnp.float32)
    ssum = jax.lax.dot_general(ones_l, w * sm0, (((1,), (0,)), ((), ())),
                               preferred_element_type=jnp.float32)
    cnt = jax.lax.dot_general(ones_l, w, (((1,), (0,)), ((), ())),
                              preferred_element_type=jnp.float32)
    inv = 1.0 / jnp.maximum(cnt, 1.0)
    score = ssum * inv
    w2 = w * inv

    for i in range(bb):
        score_ref[pl.ds(i, 1), :] = score[:, i * CP:i * CP + C]
        wi = w2[:, i * CP:i * CP + C]
        g = jax.lax.dot_general(wi, hns[i], (((0,), (0,)), ((), ())),
                                preferred_element_type=jnp.float32)
        emb_ref[i] = g


def kernel(hist_item_emb, hist_item_mask, cand_item_emb):
    B, L, D = hist_item_emb.shape
    C = cand_item_emb.shape[1]
    bb = 8
    CP = 64

    body = functools.partial(_body, bb=bb, L=L, C=C, D=D)

    out = pl.pallas_call(
        body,
        grid=(B // bb,),
        in_specs=[
            pl.BlockSpec((bb, L, D), lambda i: (i, 0, 0)),
            pl.BlockSpec((bb, L), lambda i: (i, 0)),
            pl.BlockSpec((bb, C, D), lambda i: (i, 0, 0)),
        ],
        out_specs=[
            pl.BlockSpec((bb, C), lambda i: (i, 0)),
            pl.BlockSpec((bb, C, D), lambda i: (i, 0, 0)),
        ],
        out_shape=[
            jax.ShapeDtypeStruct((B, C), jnp.float32),
            jax.ShapeDtypeStruct((B, C, D), jnp.float32),
        ],
        scratch_shapes=[pltpu.VMEM((L, bb * CP), jnp.float32)],
    )(hist_item_emb, hist_item_mask, cand_item_emb)
    return (out[0], out[1])
